# baseline (speedup 1.0000x reference)
import functools

import jax
import jax.numpy as jnp
from jax import lax
from jax.experimental import pallas as pl
from jax.experimental.pallas import tpu as pltpu
from jax.experimental.pallas import tpu_sc as plsc

NC, NS = 2, 16
NW = NC * NS
N_NODES = 10000
N_PAD = 10240
D_H = 64
DEG_W = 16
BLK = 80
BLK_A = 200


def _mesh():
    return plsc.VectorSubcoreMesh(
        core_axis_name="c", subcore_axis_name="s", num_cores=NC, num_subcores=NS
    )


def _sc_degree(dst3):
    nblk = dst3.shape[1]
    rows_per_tile = N_PAD // NS

    @functools.partial(
        pl.kernel,
        out_type=jax.ShapeDtypeStruct((NC, N_PAD, DEG_W), jnp.float32),
        mesh=_mesh(),
        compiler_params=pltpu.CompilerParams(use_tc_tiling_on_sc=False),
        scratch_types=[
            pltpu.VMEM((nblk, BLK), jnp.int32),
            pltpu.VMEM((BLK, DEG_W), jnp.float32),
            pltpu.VMEM((rows_per_tile, DEG_W), jnp.float32),
            pltpu.VMEM_SHARED((N_PAD, DEG_W), jnp.float32),
            pltpu.SemaphoreType.DMA,
        ],
    )
    def k(dst_hbm, out_hbm, didx, ones_v, zbuf, acc, sem):
        c = lax.axis_index("c")
        s = lax.axis_index("s")
        tid = c * NS + s

        def fill_ones(i, _):
            ones_v[i] = jnp.ones((DEG_W,), jnp.float32)
            return 0

        lax.fori_loop(0, BLK, fill_ones, 0)

        def fill_zeros(i, _):
            zbuf[i] = jnp.zeros((DEG_W,), jnp.float32)
            return 0

        lax.fori_loop(0, rows_per_tile, fill_zeros, 0)
        pltpu.sync_copy(dst_hbm.at[tid], didx)
        pltpu.sync_copy(zbuf, acc.at[pl.ds(s * rows_per_tile, rows_per_tile)])
        plsc.subcore_barrier()

        def body(i, _):
            pltpu.async_copy(ones_v, acc.at[didx.at[i]], sem, add=True)
            return 0

        lax.fori_loop(0, nblk, body, 0)

        def drain(i, _):
            pltpu.make_async_copy(ones_v, acc.at[didx.at[i]], sem).wait()
            return 0

        lax.fori_loop(0, nblk, drain, 0)
        plsc.subcore_barrier()
        pltpu.sync_copy(
            acc.at[pl.ds(s * rows_per_tile, rows_per_tile)],
            out_hbm.at[c, pl.ds(s * rows_per_tile, rows_per_tile)],
        )

    return k(dst3)


NBUF = 5


def _sc_edge_agg(m, src3, dst3):
    nblk = src3.shape[1]
    rows_per_tile = N_PAD // NS

    @functools.partial(
        pl.kernel,
        out_type=jax.ShapeDtypeStruct((NC, N_PAD, D_H), jnp.float32),
        mesh=_mesh(),
        compiler_params=pltpu.CompilerParams(use_tc_tiling_on_sc=False),
        scratch_types=[
            pltpu.VMEM((nblk, BLK_A), jnp.int32),
            pltpu.VMEM((nblk, BLK_A), jnp.int32),
            pltpu.VMEM((NBUF, BLK_A, D_H), jnp.float32),
            pltpu.VMEM_SHARED((N_PAD, D_H), jnp.float32),
            [pltpu.SemaphoreType.DMA] * NBUF,
            [pltpu.SemaphoreType.DMA] * NBUF,
        ],
    )
    def k(m_hbm, src_hbm, dst_hbm, out_hbm, sidx, didx, rows, acc,
          sem_g, sem_s):
        c = lax.axis_index("c")
        s = lax.axis_index("s")
        tid = c * NS + s

        def fill_zeros(i, _):
            rows[0, i] = jnp.zeros((D_H,), jnp.float32)
            return 0

        lax.fori_loop(0, BLK_A, fill_zeros, 0)
        pltpu.sync_copy(src_hbm.at[tid], sidx)
        pltpu.sync_copy(dst_hbm.at[tid], didx)
        base0 = s * rows_per_tile
        done = 0
        while done < rows_per_tile:
            n = min(BLK_A, rows_per_tile - done)
            pltpu.sync_copy(rows.at[0, pl.ds(0, n)],
                            acc.at[pl.ds(base0 + done, n)])
            done += n
        plsc.subcore_barrier()

        def gather(i, b):
            return pltpu.async_copy(m_hbm.at[sidx.at[i]], rows.at[b], sem_g[b])

        def scat(i, b):
            return pltpu.async_copy(rows.at[b], acc.at[didx.at[i]], sem_s[b],
                                    add=True)

        for b in range(3):
            gather(b, b)

        def outer(g, _):
            for b in range(NBUF):
                i = g * NBUF + b
                pltpu.make_async_copy(
                    m_hbm.at[sidx.at[i]], rows.at[b], sem_g[b]).wait()
                scat(i, b)
                bn = (b + 3) % NBUF

                @pl.when(i >= 2)
                def _():
                    pltpu.make_async_copy(
                        rows.at[bn], acc.at[didx.at[i - 2]], sem_s[bn]).wait()

                @pl.when(i + 3 < nblk)
                def _():
                    gather(i + 3, bn)
            return 0

        lax.fori_loop(0, nblk // NBUF, outer, 0)
        for i in (nblk - 2, nblk - 1):
            b = i % NBUF
            pltpu.make_async_copy(
                rows.at[b], acc.at[didx.at[i]], sem_s[b]).wait()
        plsc.subcore_barrier()
        pltpu.sync_copy(
            acc.at[pl.ds(s * rows_per_tile, rows_per_tile)],
            out_hbm.at[c, pl.ds(s * rows_per_tile, rows_per_tile)],
        )

    return k(m, src3, dst3)


def _sc_pair_gather(P, Q, src3, dst3):
    nblk = src3.shape[1]
    chunk = nblk * BLK
    E = NW * chunk

    @functools.partial(
        pl.kernel,
        out_type=jax.ShapeDtypeStruct((E, 2 * D_H), jnp.bfloat16),
        mesh=_mesh(),
        compiler_params=pltpu.CompilerParams(use_tc_tiling_on_sc=False),
        scratch_types=[
            pltpu.VMEM((nblk, BLK), jnp.int32),
            pltpu.VMEM((nblk, BLK), jnp.int32),
            pltpu.VMEM((NBUF, BLK, D_H), jnp.bfloat16),
            pltpu.VMEM((NBUF, BLK, D_H), jnp.bfloat16),
            [pltpu.SemaphoreType.DMA] * NBUF,
            [pltpu.SemaphoreType.DMA] * NBUF,
        ],
    )
    def k(p_hbm, q_hbm, src_hbm, dst_hbm, rpq_hbm, sidx, didx,
          ra, rb, sem_g, sem_w):
        c = lax.axis_index("c")
        s = lax.axis_index("s")
        tid = c * NS + s

        pltpu.sync_copy(src_hbm.at[tid], sidx)
        pltpu.sync_copy(dst_hbm.at[tid], didx)

        def gather(i, b):
            pltpu.async_copy(p_hbm.at[sidx.at[i]], ra.at[b], sem_g[b])
            pltpu.async_copy(q_hbm.at[didx.at[i]], rb.at[b], sem_g[b])

        def wait_gather(i, b):
            pltpu.make_async_copy(p_hbm.at[sidx.at[i]], ra.at[b],
                                  sem_g[b]).wait()
            pltpu.make_async_copy(q_hbm.at[didx.at[i]], rb.at[b],
                                  sem_g[b]).wait()

        def write(i, b):
            base = tid * chunk + i * BLK
            pltpu.async_copy(
                ra.at[b], rpq_hbm.at[pl.ds(base, BLK), pl.ds(0, D_H)],
                sem_w[b])
            pltpu.async_copy(
                rb.at[b], rpq_hbm.at[pl.ds(base, BLK), pl.ds(D_H, D_H)],
                sem_w[b])

        def wait_write(i, b):
            base = tid * chunk + i * BLK
            pltpu.make_async_copy(
                ra.at[b], rpq_hbm.at[pl.ds(base, BLK), pl.ds(0, D_H)],
                sem_w[b]).wait()
            pltpu.make_async_copy(
                rb.at[b], rpq_hbm.at[pl.ds(base, BLK), pl.ds(D_H, D_H)],
                sem_w[b]).wait()

        for b in range(3):
            gather(b, b)

        def outer(g, _):
            for b in range(NBUF):
                i = g * NBUF + b
                wait_gather(i, b)
                write(i, b)
                bn = (b + 3) % NBUF

                @pl.when(i >= 2)
                def _():
                    wait_write(i - 2, bn)

                @pl.when(i + 3 < nblk)
                def _():
                    gather(i + 3, bn)
            return 0

        lax.fori_loop(0, nblk // NBUF, outer, 0)
        for i in (nblk - 2, nblk - 1):
            wait_write(i, i % NBUF)

    return k(P, Q, src3, dst3)


def _tc_dis_scale(parts, x, W1):
    def body(p_ref, x_ref, w_ref, dis_ref, m_ref):
        p = p_ref[...]
        deg = p[0, :N_NODES, 0:1] + p[1, :N_NODES, 0:1] + 1.0
        dis64 = jnp.broadcast_to(lax.rsqrt(deg), (N_NODES, D_H))
        dis_ref[...] = dis64
        m_ref[...] = jnp.dot(x_ref[...], w_ref[...],
                             preferred_element_type=jnp.float32) * dis64

    return pl.pallas_call(
        body,
        out_shape=(
            jax.ShapeDtypeStruct((N_NODES, D_H), jnp.float32),
            jax.ShapeDtypeStruct((N_NODES, D_H), jnp.float32),
        ),
    )(parts, x, W1)


def _tc_layer(parts, m_prev, dis64, b_row, W_next):
    def body(p_ref, m_ref, d_ref, b_ref, w_ref, o_ref):
        p = p_ref[...]
        d = d_ref[...]
        h = jnp.maximum(
            (p[0, :N_NODES] + p[1, :N_NODES] + m_ref[...]) * d + b_ref[...], 0.0)
        o_ref[...] = jnp.dot(h, w_ref[...],
                             preferred_element_type=jnp.float32) * d

    return pl.pallas_call(
        body,
        out_shape=jax.ShapeDtypeStruct((N_NODES, D_H), jnp.float32),
    )(parts, m_prev, dis64, b_row, W_next)


def _tc_final_nodes(parts, m_prev, dis64, b_row, lin1_W, lin1_b_row):
    def body(p_ref, m_ref, d_ref, b_ref, w_ref, lb_ref, P_ref, Q_ref):
        p = p_ref[...]
        w = w_ref[...]
        h4 = jnp.maximum(
            (p[0, :N_NODES] + p[1, :N_NODES] + m_ref[...]) * d_ref[...]
            + b_ref[...], 0.0)
        P_ref[...] = jnp.dot(
            h4, w[:D_H], preferred_element_type=jnp.float32
        ).astype(jnp.bfloat16)
        Q_ref[...] = (jnp.dot(h4, w[D_H:], preferred_element_type=jnp.float32)
                      + lb_ref[...]).astype(jnp.bfloat16)

    return pl.pallas_call(
        body,
        out_shape=(
            jax.ShapeDtypeStruct((N_NODES, D_H), jnp.bfloat16),
            jax.ShapeDtypeStruct((N_NODES, D_H), jnp.bfloat16),
        ),
    )(parts, m_prev, dis64, b_row, lin1_W, lin1_b_row)


def _tc_head(rpq, lin2_W, lin2_b_row):
    E = rpq.shape[0]
    BE = 8000
    n_cls = lin2_W.shape[1]

    def body(a_ref, w_ref, bias_ref, o_ref):
        a = a_ref[...].astype(jnp.float32)
        r = jnp.maximum(a[:, :D_H] + a[:, D_H:], 0.0)
        logits = jnp.dot(r, w_ref[...],
                         preferred_element_type=jnp.float32) + bias_ref[...]
        mx = jnp.max(logits, axis=1, keepdims=True)
        lse = jnp.log(jnp.sum(jnp.exp(logits - mx), axis=1, keepdims=True)) + mx
        o_ref[...] = logits - lse

    return pl.pallas_call(
        body,
        grid=(E // BE,),
        in_specs=[
            pl.BlockSpec((BE, 2 * D_H), lambda i: (i, 0)),
            pl.BlockSpec((D_H, n_cls), lambda i: (0, 0)),
            pl.BlockSpec((1, n_cls), lambda i: (0, 0)),
        ],
        out_specs=pl.BlockSpec((BE, n_cls), lambda i: (i, 0)),
        out_shape=jax.ShapeDtypeStruct((E, n_cls), jnp.float32),
    )(rpq, lin2_W, lin2_b_row)


def kernel(x, edge_index, W1, b1, W2, b2, W3, b3, lin1_W, lin1_b, lin2_W, lin2_b):
    ei = edge_index.astype(jnp.int32)
    E = ei.shape[1]
    nblk = E // (NW * BLK)
    nblk_a = E // (NW * BLK_A)
    src = ei[0].reshape(NW, nblk, BLK)
    dst = ei[1].reshape(NW, nblk, BLK)
    src_a = ei[0].reshape(NW, nblk_a, BLK_A)
    dst_a = ei[1].reshape(NW, nblk_a, BLK_A)

    parts_deg = _sc_degree(dst)
    dis64, m1 = _tc_dis_scale(parts_deg, x, W1)

    p = _sc_edge_agg(m1, src_a, dst_a)
    m2 = _tc_layer(p, m1, dis64, b1.reshape(1, -1), W2)
    p = _sc_edge_agg(m2, src_a, dst_a)
    m3 = _tc_layer(p, m2, dis64, b2.reshape(1, -1), W3)
    p = _sc_edge_agg(m3, src_a, dst_a)

    P, Q = _tc_final_nodes(
        p, m3, dis64, b3.reshape(1, -1), lin1_W, lin1_b.reshape(1, -1),
    )
    rpq = _sc_pair_gather(P, Q, src, dst)
    return _tc_head(rpq, lin2_W, lin2_b.reshape(1, -1))


# speedup vs baseline: 1.4080x; 1.4080x over previous
import functools

import jax
import jax.numpy as jnp
from jax import lax
from jax.experimental import pallas as pl
from jax.experimental.pallas import tpu as pltpu
from jax.experimental.pallas import tpu_sc as plsc

NC, NS = 2, 16
NW = NC * NS
N_NODES = 10000
N_PAD = 10240
D_H = 64
DEG_W = 16
BLK = 80
BLK_A = 200


def _mesh():
    return plsc.VectorSubcoreMesh(
        core_axis_name="c", subcore_axis_name="s", num_cores=NC, num_subcores=NS
    )


def _sc_degree(dst3):
    nblk = dst3.shape[1]
    rows_per_tile = N_PAD // NS

    @functools.partial(
        pl.kernel,
        out_type=jax.ShapeDtypeStruct((NC, N_PAD, DEG_W), jnp.float32),
        mesh=_mesh(),
        compiler_params=pltpu.CompilerParams(use_tc_tiling_on_sc=False),
        scratch_types=[
            pltpu.VMEM((nblk, BLK), jnp.int32),
            pltpu.VMEM((BLK, DEG_W), jnp.float32),
            pltpu.VMEM((rows_per_tile, DEG_W), jnp.float32),
            pltpu.VMEM_SHARED((N_PAD, DEG_W), jnp.float32),
            pltpu.SemaphoreType.DMA,
        ],
    )
    def k(dst_hbm, out_hbm, didx, ones_v, zbuf, acc, sem):
        c = lax.axis_index("c")
        s = lax.axis_index("s")
        tid = c * NS + s

        def fill_ones(i, _):
            ones_v[i] = jnp.ones((DEG_W,), jnp.float32)
            return 0

        lax.fori_loop(0, BLK, fill_ones, 0)

        def fill_zeros(i, _):
            zbuf[i] = jnp.zeros((DEG_W,), jnp.float32)
            return 0

        lax.fori_loop(0, rows_per_tile, fill_zeros, 0)
        pltpu.sync_copy(dst_hbm.at[tid], didx)
        pltpu.sync_copy(zbuf, acc.at[pl.ds(s * rows_per_tile, rows_per_tile)])
        plsc.subcore_barrier()

        def body(i, _):
            pltpu.async_copy(ones_v, acc.at[didx.at[i]], sem, add=True)
            return 0

        lax.fori_loop(0, nblk, body, 0)

        def drain(i, _):
            pltpu.make_async_copy(ones_v, acc.at[didx.at[i]], sem).wait()
            return 0

        lax.fori_loop(0, nblk, drain, 0)
        plsc.subcore_barrier()
        pltpu.sync_copy(
            acc.at[pl.ds(s * rows_per_tile, rows_per_tile)],
            out_hbm.at[c, pl.ds(s * rows_per_tile, rows_per_tile)],
        )

    return k(dst3)


NBUF = 5


def _sc_edge_agg(m, src3, dst3):
    nblk = src3.shape[1]
    rows_per_tile = N_PAD // NS

    @functools.partial(
        pl.kernel,
        out_type=jax.ShapeDtypeStruct((NC, N_PAD, D_H), jnp.float32),
        mesh=_mesh(),
        compiler_params=pltpu.CompilerParams(use_tc_tiling_on_sc=False),
        scratch_types=[
            pltpu.VMEM((nblk, BLK_A), jnp.int32),
            pltpu.VMEM((nblk, BLK_A), jnp.int32),
            pltpu.VMEM((NBUF, BLK_A, D_H), jnp.float32),
            pltpu.VMEM_SHARED((N_PAD, D_H), jnp.float32),
            [pltpu.SemaphoreType.DMA] * NBUF,
            [pltpu.SemaphoreType.DMA] * NBUF,
        ],
    )
    def k(m_hbm, src_hbm, dst_hbm, out_hbm, sidx, didx, rows, acc,
          sem_g, sem_s):
        c = lax.axis_index("c")
        s = lax.axis_index("s")
        tid = c * NS + s

        def fill_zeros(i, _):
            rows[0, i] = jnp.zeros((D_H,), jnp.float32)
            return 0

        lax.fori_loop(0, BLK_A, fill_zeros, 0)
        pltpu.sync_copy(src_hbm.at[tid], sidx)
        pltpu.sync_copy(dst_hbm.at[tid], didx)
        base0 = s * rows_per_tile
        done = 0
        while done < rows_per_tile:
            n = min(BLK_A, rows_per_tile - done)
            pltpu.sync_copy(rows.at[0, pl.ds(0, n)],
                            acc.at[pl.ds(base0 + done, n)])
            done += n
        plsc.subcore_barrier()

        def gather(i, b):
            return pltpu.async_copy(m_hbm.at[sidx.at[i]], rows.at[b], sem_g[b])

        def scat(i, b):
            return pltpu.async_copy(rows.at[b], acc.at[didx.at[i]], sem_s[b],
                                    add=True)

        for b in range(3):
            gather(b, b)

        def outer(g, _):
            for b in range(NBUF):
                i = g * NBUF + b
                pltpu.make_async_copy(
                    m_hbm.at[sidx.at[i]], rows.at[b], sem_g[b]).wait()
                scat(i, b)
                bn = (b + 3) % NBUF

                @pl.when(i >= 2)
                def _():
                    pltpu.make_async_copy(
                        rows.at[bn], acc.at[didx.at[i - 2]], sem_s[bn]).wait()

                @pl.when(i + 3 < nblk)
                def _():
                    gather(i + 3, bn)
            return 0

        lax.fori_loop(0, nblk // NBUF, outer, 0)
        for i in (nblk - 2, nblk - 1):
            b = i % NBUF
            pltpu.make_async_copy(
                rows.at[b], acc.at[didx.at[i]], sem_s[b]).wait()
        plsc.subcore_barrier()
        pltpu.sync_copy(
            acc.at[pl.ds(s * rows_per_tile, rows_per_tile)],
            out_hbm.at[c, pl.ds(s * rows_per_tile, rows_per_tile)],
        )

    return k(m, src3, dst3)


def _sc_pair_gather(P, Q, src3, dst3):
    nblk = src3.shape[1]
    blk = src3.shape[2]
    chunk = nblk * blk
    E = NW * chunk

    @functools.partial(
        pl.kernel,
        out_type=jax.ShapeDtypeStruct((E, 2 * D_H), jnp.float32),
        mesh=_mesh(),
        compiler_params=pltpu.CompilerParams(use_tc_tiling_on_sc=False),
        scratch_types=[
            pltpu.VMEM((nblk, blk), jnp.int32),
            pltpu.VMEM((nblk, blk), jnp.int32),
            pltpu.VMEM((NBUF, blk, D_H), jnp.float32),
            pltpu.VMEM((NBUF, blk, D_H), jnp.float32),
            [pltpu.SemaphoreType.DMA] * NBUF,
            [pltpu.SemaphoreType.DMA] * NBUF,
        ],
    )
    def k(p_hbm, q_hbm, src_hbm, dst_hbm, rpq_hbm, sidx, didx,
          ra, rb, sem_g, sem_w):
        c = lax.axis_index("c")
        s = lax.axis_index("s")
        tid = c * NS + s

        pltpu.sync_copy(src_hbm.at[tid], sidx)
        pltpu.sync_copy(dst_hbm.at[tid], didx)

        def gather(i, b):
            pltpu.async_copy(p_hbm.at[sidx.at[i]], ra.at[b], sem_g[b])
            pltpu.async_copy(q_hbm.at[didx.at[i]], rb.at[b], sem_g[b])

        def wait_gather(i, b):
            pltpu.make_async_copy(p_hbm.at[sidx.at[i]], ra.at[b],
                                  sem_g[b]).wait()
            pltpu.make_async_copy(q_hbm.at[didx.at[i]], rb.at[b],
                                  sem_g[b]).wait()

        def write(i, b):
            base = tid * chunk + i * blk
            pltpu.async_copy(
                ra.at[b], rpq_hbm.at[pl.ds(base, blk), pl.ds(0, D_H)],
                sem_w[b])
            pltpu.async_copy(
                rb.at[b], rpq_hbm.at[pl.ds(base, blk), pl.ds(D_H, D_H)],
                sem_w[b])

        def wait_write(i, b):
            base = tid * chunk + i * blk
            pltpu.make_async_copy(
                ra.at[b], rpq_hbm.at[pl.ds(base, blk), pl.ds(0, D_H)],
                sem_w[b]).wait()
            pltpu.make_async_copy(
                rb.at[b], rpq_hbm.at[pl.ds(base, blk), pl.ds(D_H, D_H)],
                sem_w[b]).wait()

        for b in range(3):
            gather(b, b)

        def outer(g, _):
            for b in range(NBUF):
                i = g * NBUF + b
                wait_gather(i, b)
                write(i, b)
                bn = (b + 3) % NBUF

                @pl.when(i >= 2)
                def _():
                    wait_write(i - 2, bn)

                @pl.when(i + 3 < nblk)
                def _():
                    gather(i + 3, bn)
            return 0

        lax.fori_loop(0, nblk // NBUF, outer, 0)
        for i in (nblk - 2, nblk - 1):
            wait_write(i, i % NBUF)

    return k(P, Q, src3, dst3)


def _tc_dis_scale(parts, x, W1):
    def body(p_ref, x_ref, w_ref, dis_ref, m_ref):
        p = p_ref[...]
        deg = p[0, :N_NODES, 0:1] + p[1, :N_NODES, 0:1] + 1.0
        dis64 = jnp.broadcast_to(lax.rsqrt(deg), (N_NODES, D_H))
        dis_ref[...] = dis64
        m_ref[...] = jnp.dot(x_ref[...], w_ref[...],
                             preferred_element_type=jnp.float32) * dis64

    return pl.pallas_call(
        body,
        out_shape=(
            jax.ShapeDtypeStruct((N_NODES, D_H), jnp.float32),
            jax.ShapeDtypeStruct((N_NODES, D_H), jnp.float32),
        ),
    )(parts, x, W1)


def _tc_layer(parts, m_prev, dis64, b_row, W_next):
    def body(p_ref, m_ref, d_ref, b_ref, w_ref, o_ref):
        p = p_ref[...]
        d = d_ref[...]
        h = jnp.maximum(
            (p[0, :N_NODES] + p[1, :N_NODES] + m_ref[...]) * d + b_ref[...], 0.0)
        o_ref[...] = jnp.dot(h, w_ref[...],
                             preferred_element_type=jnp.float32) * d

    return pl.pallas_call(
        body,
        out_shape=jax.ShapeDtypeStruct((N_NODES, D_H), jnp.float32),
    )(parts, m_prev, dis64, b_row, W_next)


def _tc_final_nodes(parts, m_prev, dis64, b_row, lin1_W, lin1_b_row):
    def body(p_ref, m_ref, d_ref, b_ref, w_ref, lb_ref, P_ref, Q_ref):
        p = p_ref[...]
        w = w_ref[...]
        h4 = jnp.maximum(
            (p[0, :N_NODES] + p[1, :N_NODES] + m_ref[...]) * d_ref[...]
            + b_ref[...], 0.0)
        P_ref[...] = jnp.dot(h4, w[:D_H],
                             preferred_element_type=jnp.float32)
        Q_ref[...] = jnp.dot(h4, w[D_H:],
                             preferred_element_type=jnp.float32) + lb_ref[...]

    return pl.pallas_call(
        body,
        out_shape=(
            jax.ShapeDtypeStruct((N_NODES, D_H), jnp.float32),
            jax.ShapeDtypeStruct((N_NODES, D_H), jnp.float32),
        ),
    )(parts, m_prev, dis64, b_row, lin1_W, lin1_b_row)


def _tc_head(rpq, lin2_W, lin2_b_row):
    E = rpq.shape[0]
    BE = 8000
    n_cls = lin2_W.shape[1]

    def body(a_ref, w_ref, bias_ref, o_ref):
        a = a_ref[...]
        r = jnp.maximum(a[:, :D_H] + a[:, D_H:], 0.0)
        logits = jnp.dot(r, w_ref[...],
                         preferred_element_type=jnp.float32) + bias_ref[...]
        mx = jnp.max(logits, axis=1, keepdims=True)
        lse = jnp.log(jnp.sum(jnp.exp(logits - mx), axis=1, keepdims=True)) + mx
        o_ref[...] = logits - lse

    return pl.pallas_call(
        body,
        grid=(E // BE,),
        in_specs=[
            pl.BlockSpec((BE, 2 * D_H), lambda i: (i, 0)),
            pl.BlockSpec((D_H, n_cls), lambda i: (0, 0)),
            pl.BlockSpec((1, n_cls), lambda i: (0, 0)),
        ],
        out_specs=pl.BlockSpec((BE, n_cls), lambda i: (i, 0)),
        out_shape=jax.ShapeDtypeStruct((E, n_cls), jnp.float32),
    )(rpq, lin2_W, lin2_b_row)


def kernel(x, edge_index, W1, b1, W2, b2, W3, b3, lin1_W, lin1_b, lin2_W, lin2_b):
    ei = edge_index.astype(jnp.int32)
    E = ei.shape[1]
    nblk = E // (NW * BLK)
    nblk_a = E // (NW * BLK_A)
    src = ei[0].reshape(NW, nblk, BLK)
    dst = ei[1].reshape(NW, nblk, BLK)
    src_a = ei[0].reshape(NW, nblk_a, BLK_A)
    dst_a = ei[1].reshape(NW, nblk_a, BLK_A)

    parts_deg = _sc_degree(dst)
    dis64, m1 = _tc_dis_scale(parts_deg, x, W1)

    p = _sc_edge_agg(m1, src_a, dst_a)
    m2 = _tc_layer(p, m1, dis64, b1.reshape(1, -1), W2)
    p = _sc_edge_agg(m2, src_a, dst_a)
    m3 = _tc_layer(p, m2, dis64, b2.reshape(1, -1), W3)
    p = _sc_edge_agg(m3, src_a, dst_a)

    P, Q = _tc_final_nodes(
        p, m3, dis64, b3.reshape(1, -1), lin1_W, lin1_b.reshape(1, -1),
    )
    Eh = E // 2
    BLK_P = 100
    nblk_p = Eh // (NW * BLK_P)
    lb_row = lin2_b.reshape(1, -1)
    outs = []
    for lo in (0, Eh):
        src_h = lax.dynamic_slice_in_dim(ei[0], lo, Eh).reshape(
            NW, nblk_p, BLK_P)
        dst_h = lax.dynamic_slice_in_dim(ei[1], lo, Eh).reshape(
            NW, nblk_p, BLK_P)
        rpq = _sc_pair_gather(P, Q, src_h, dst_h)
        outs.append(_tc_head(rpq, lin2_W, lb_row))
    return jnp.concatenate(outs, axis=0)


# speedup vs baseline: 1.4119x; 1.0027x over previous
import functools

import jax
import jax.numpy as jnp
from jax import lax
from jax.experimental import pallas as pl
from jax.experimental.pallas import tpu as pltpu
from jax.experimental.pallas import tpu_sc as plsc

NC, NS = 2, 16
NW = NC * NS
N_NODES = 10000
N_PAD = 10240
D_H = 64
DEG_W = 16
BLK = 80
BLK_A = 200


def _mesh():
    return plsc.VectorSubcoreMesh(
        core_axis_name="c", subcore_axis_name="s", num_cores=NC, num_subcores=NS
    )


def _sc_degree(dst3):
    nblk = dst3.shape[1]
    rows_per_tile = N_PAD // NS

    @functools.partial(
        pl.kernel,
        out_type=jax.ShapeDtypeStruct((NC, N_PAD, DEG_W), jnp.float32),
        mesh=_mesh(),
        compiler_params=pltpu.CompilerParams(use_tc_tiling_on_sc=False),
        scratch_types=[
            pltpu.VMEM((nblk, BLK), jnp.int32),
            pltpu.VMEM((BLK, DEG_W), jnp.float32),
            pltpu.VMEM((rows_per_tile, DEG_W), jnp.float32),
            pltpu.VMEM_SHARED((N_PAD, DEG_W), jnp.float32),
            pltpu.SemaphoreType.DMA,
        ],
    )
    def k(dst_hbm, out_hbm, didx, ones_v, zbuf, acc, sem):
        c = lax.axis_index("c")
        s = lax.axis_index("s")
        tid = c * NS + s

        def fill_ones(i, _):
            ones_v[i] = jnp.ones((DEG_W,), jnp.float32)
            return 0

        lax.fori_loop(0, BLK, fill_ones, 0)

        def fill_zeros(i, _):
            zbuf[i] = jnp.zeros((DEG_W,), jnp.float32)
            return 0

        lax.fori_loop(0, rows_per_tile, fill_zeros, 0)
        pltpu.sync_copy(dst_hbm.at[tid], didx)
        pltpu.sync_copy(zbuf, acc.at[pl.ds(s * rows_per_tile, rows_per_tile)])
        plsc.subcore_barrier()

        def body(i, _):
            pltpu.async_copy(ones_v, acc.at[didx.at[i]], sem, add=True)
            return 0

        lax.fori_loop(0, nblk, body, 0)

        def drain(i, _):
            pltpu.make_async_copy(ones_v, acc.at[didx.at[i]], sem).wait()
            return 0

        lax.fori_loop(0, nblk, drain, 0)
        plsc.subcore_barrier()
        pltpu.sync_copy(
            acc.at[pl.ds(s * rows_per_tile, rows_per_tile)],
            out_hbm.at[c, pl.ds(s * rows_per_tile, rows_per_tile)],
        )

    return k(dst3)


NBUF = 5


def _sc_edge_agg(m, src3, dst3):
    nblk = src3.shape[1]
    rows_per_tile = N_PAD // NS

    @functools.partial(
        pl.kernel,
        out_type=jax.ShapeDtypeStruct((NC, N_PAD, D_H), jnp.float32),
        mesh=_mesh(),
        compiler_params=pltpu.CompilerParams(use_tc_tiling_on_sc=False),
        scratch_types=[
            pltpu.VMEM((nblk, BLK_A), jnp.int32),
            pltpu.VMEM((nblk, BLK_A), jnp.int32),
            pltpu.VMEM((NBUF, BLK_A, D_H), jnp.float32),
            pltpu.VMEM_SHARED((N_PAD, D_H), jnp.float32),
            [pltpu.SemaphoreType.DMA] * NBUF,
            [pltpu.SemaphoreType.DMA] * NBUF,
        ],
    )
    def k(m_hbm, src_hbm, dst_hbm, out_hbm, sidx, didx, rows, acc,
          sem_g, sem_s):
        c = lax.axis_index("c")
        s = lax.axis_index("s")
        tid = c * NS + s

        def fill_zeros(i, _):
            rows[0, i] = jnp.zeros((D_H,), jnp.float32)
            return 0

        lax.fori_loop(0, BLK_A, fill_zeros, 0)
        pltpu.sync_copy(src_hbm.at[tid], sidx)
        pltpu.sync_copy(dst_hbm.at[tid], didx)
        base0 = s * rows_per_tile
        done = 0
        while done < rows_per_tile:
            n = min(BLK_A, rows_per_tile - done)
            pltpu.sync_copy(rows.at[0, pl.ds(0, n)],
                            acc.at[pl.ds(base0 + done, n)])
            done += n
        plsc.subcore_barrier()

        def gather(i, b):
            return pltpu.async_copy(m_hbm.at[sidx.at[i]], rows.at[b], sem_g[b])

        def scat(i, b):
            return pltpu.async_copy(rows.at[b], acc.at[didx.at[i]], sem_s[b],
                                    add=True)

        for b in range(3):
            gather(b, b)

        def outer(g, _):
            for b in range(NBUF):
                i = g * NBUF + b
                pltpu.make_async_copy(
                    m_hbm.at[sidx.at[i]], rows.at[b], sem_g[b]).wait()
                scat(i, b)
                bn = (b + 3) % NBUF

                @pl.when(i >= 2)
                def _():
                    pltpu.make_async_copy(
                        rows.at[bn], acc.at[didx.at[i - 2]], sem_s[bn]).wait()

                @pl.when(i + 3 < nblk)
                def _():
                    gather(i + 3, bn)
            return 0

        lax.fori_loop(0, nblk // NBUF, outer, 0)
        for i in (nblk - 2, nblk - 1):
            b = i % NBUF
            pltpu.make_async_copy(
                rows.at[b], acc.at[didx.at[i]], sem_s[b]).wait()
        plsc.subcore_barrier()
        pltpu.sync_copy(
            acc.at[pl.ds(s * rows_per_tile, rows_per_tile)],
            out_hbm.at[c, pl.ds(s * rows_per_tile, rows_per_tile)],
        )

    return k(m, src3, dst3)


def _sc_pair_gather(P, Q, src3, dst3):
    nblk = src3.shape[1]
    chunk = nblk * BLK
    E = NW * chunk

    @functools.partial(
        pl.kernel,
        out_type=jax.ShapeDtypeStruct((E, 2 * D_H), jnp.float32),
        mesh=_mesh(),
        compiler_params=pltpu.CompilerParams(use_tc_tiling_on_sc=False),
        scratch_types=[
            pltpu.VMEM((nblk, BLK), jnp.int32),
            pltpu.VMEM((nblk, BLK), jnp.int32),
            pltpu.VMEM((NBUF, BLK, D_H), jnp.float32),
            pltpu.VMEM((NBUF, BLK, D_H), jnp.float32),
            [pltpu.SemaphoreType.DMA] * NBUF,
            [pltpu.SemaphoreType.DMA] * NBUF,
        ],
    )
    def k(p_hbm, q_hbm, src_hbm, dst_hbm, rpq_hbm, sidx, didx,
          ra, rb, sem_g, sem_w):
        c = lax.axis_index("c")
        s = lax.axis_index("s")
        tid = c * NS + s

        pltpu.sync_copy(src_hbm.at[tid], sidx)
        pltpu.sync_copy(dst_hbm.at[tid], didx)

        def gather(i, b):
            pltpu.async_copy(p_hbm.at[sidx.at[i]], ra.at[b], sem_g[b])
            pltpu.async_copy(q_hbm.at[didx.at[i]], rb.at[b], sem_g[b])

        def wait_gather(i, b):
            pltpu.make_async_copy(p_hbm.at[sidx.at[i]], ra.at[b],
                                  sem_g[b]).wait()
            pltpu.make_async_copy(q_hbm.at[didx.at[i]], rb.at[b],
                                  sem_g[b]).wait()

        def write(i, b):
            base = tid * chunk + i * BLK
            pltpu.async_copy(
                ra.at[b], rpq_hbm.at[pl.ds(base, BLK), pl.ds(0, D_H)],
                sem_w[b])
            pltpu.async_copy(
                rb.at[b], rpq_hbm.at[pl.ds(base, BLK), pl.ds(D_H, D_H)],
                sem_w[b])

        def wait_write(i, b):
            base = tid * chunk + i * BLK
            pltpu.make_async_copy(
                ra.at[b], rpq_hbm.at[pl.ds(base, BLK), pl.ds(0, D_H)],
                sem_w[b]).wait()
            pltpu.make_async_copy(
                rb.at[b], rpq_hbm.at[pl.ds(base, BLK), pl.ds(D_H, D_H)],
                sem_w[b]).wait()

        for b in range(3):
            gather(b, b)

        def outer(g, _):
            for b in range(NBUF):
                i = g * NBUF + b
                wait_gather(i, b)
                write(i, b)
                bn = (b + 3) % NBUF

                @pl.when(i >= 2)
                def _():
                    wait_write(i - 2, bn)

                @pl.when(i + 3 < nblk)
                def _():
                    gather(i + 3, bn)
            return 0

        lax.fori_loop(0, nblk // NBUF, outer, 0)
        for i in (nblk - 2, nblk - 1):
            wait_write(i, i % NBUF)

    return k(P, Q, src3, dst3)


def _tc_dis_scale(parts, x, W1):
    def body(p_ref, x_ref, w_ref, dis_ref, m_ref):
        p = p_ref[...]
        deg = p[0, :N_NODES, 0:1] + p[1, :N_NODES, 0:1] + 1.0
        dis64 = jnp.broadcast_to(lax.rsqrt(deg), (N_NODES, D_H))
        dis_ref[...] = dis64
        m_ref[...] = jnp.dot(x_ref[...], w_ref[...],
                             preferred_element_type=jnp.float32) * dis64

    return pl.pallas_call(
        body,
        out_shape=(
            jax.ShapeDtypeStruct((N_NODES, D_H), jnp.float32),
            jax.ShapeDtypeStruct((N_NODES, D_H), jnp.float32),
        ),
    )(parts, x, W1)


def _tc_layer(parts, m_prev, dis64, b_row, W_next):
    def body(p_ref, m_ref, d_ref, b_ref, w_ref, o_ref):
        p = p_ref[...]
        d = d_ref[...]
        h = jnp.maximum(
            (p[0, :N_NODES] + p[1, :N_NODES] + m_ref[...]) * d + b_ref[...], 0.0)
        o_ref[...] = jnp.dot(h, w_ref[...],
                             preferred_element_type=jnp.float32) * d

    return pl.pallas_call(
        body,
        out_shape=jax.ShapeDtypeStruct((N_NODES, D_H), jnp.float32),
    )(parts, m_prev, dis64, b_row, W_next)


def _tc_final_nodes(parts, m_prev, dis64, b_row, lin1_W, lin1_b_row):
    def body(p_ref, m_ref, d_ref, b_ref, w_ref, lb_ref, P_ref, Q_ref):
        p = p_ref[...]
        w = w_ref[...]
        h4 = jnp.maximum(
            (p[0, :N_NODES] + p[1, :N_NODES] + m_ref[...]) * d_ref[...]
            + b_ref[...], 0.0)
        P_ref[...] = jnp.dot(h4, w[:D_H],
                             preferred_element_type=jnp.float32)
        Q_ref[...] = jnp.dot(h4, w[D_H:],
                             preferred_element_type=jnp.float32) + lb_ref[...]

    return pl.pallas_call(
        body,
        out_shape=(
            jax.ShapeDtypeStruct((N_NODES, D_H), jnp.float32),
            jax.ShapeDtypeStruct((N_NODES, D_H), jnp.float32),
        ),
    )(parts, m_prev, dis64, b_row, lin1_W, lin1_b_row)


def _tc_head(rpq, lin2_W, lin2_b_row):
    E = rpq.shape[0]
    BE = 8000
    n_cls = lin2_W.shape[1]

    def body(a_ref, w_ref, bias_ref, o_ref):
        a = a_ref[...]
        r = jnp.maximum(a[:, :D_H] + a[:, D_H:], 0.0)
        logits = jnp.dot(r, w_ref[...],
                         preferred_element_type=jnp.float32) + bias_ref[...]
        mx = jnp.max(logits, axis=1, keepdims=True)
        lse = jnp.log(jnp.sum(jnp.exp(logits - mx), axis=1, keepdims=True)) + mx
        o_ref[...] = logits - lse

    return pl.pallas_call(
        body,
        grid=(E // BE,),
        in_specs=[
            pl.BlockSpec((BE, 2 * D_H), lambda i: (i, 0)),
            pl.BlockSpec((D_H, n_cls), lambda i: (0, 0)),
            pl.BlockSpec((1, n_cls), lambda i: (0, 0)),
        ],
        out_specs=pl.BlockSpec((BE, n_cls), lambda i: (i, 0)),
        out_shape=jax.ShapeDtypeStruct((E, n_cls), jnp.float32),
    )(rpq, lin2_W, lin2_b_row)


def kernel(x, edge_index, W1, b1, W2, b2, W3, b3, lin1_W, lin1_b, lin2_W, lin2_b):
    ei = edge_index.astype(jnp.int32)
    E = ei.shape[1]
    nblk = E // (NW * BLK)
    nblk_a = E // (NW * BLK_A)
    src = ei[0].reshape(NW, nblk, BLK)
    dst = ei[1].reshape(NW, nblk, BLK)
    src_a = ei[0].reshape(NW, nblk_a, BLK_A)
    dst_a = ei[1].reshape(NW, nblk_a, BLK_A)

    parts_deg = _sc_degree(dst)
    dis64, m1 = _tc_dis_scale(parts_deg, x, W1)

    p = _sc_edge_agg(m1, src_a, dst_a)
    m2 = _tc_layer(p, m1, dis64, b1.reshape(1, -1), W2)
    p = _sc_edge_agg(m2, src_a, dst_a)
    m3 = _tc_layer(p, m2, dis64, b2.reshape(1, -1), W3)
    p = _sc_edge_agg(m3, src_a, dst_a)

    P, Q = _tc_final_nodes(
        p, m3, dis64, b3.reshape(1, -1), lin1_W, lin1_b.reshape(1, -1),
    )
    rpq = _sc_pair_gather(P, Q, src, dst)
    return _tc_head(rpq, lin2_W, lin2_b.reshape(1, -1))
